# Initial kernel scaffold; baseline (speedup 1.0000x reference)
import jax
import jax.numpy as jnp
from jax.experimental import pallas as pl


def kernel(x, conv_w, conv_b):
    raise NotImplementedError("write your pallas kernel here")


# speedup vs baseline: 4.3442x; 4.3442x over previous
import functools

import jax
import jax.numpy as jnp
from jax import lax
from jax.experimental import pallas as pl
from jax.experimental.pallas import tpu as pltpu


def _dn_kernel(x_ref, w_ref, b_ref, o_ref, *, H, W):
    x = x_ref[0]
    C, HW = x.shape

    lane = lax.broadcasted_iota(jnp.int32, (C, HW), 1)
    col = lane % W
    row = lane // W

    zc = jnp.zeros((C, 1), x.dtype)
    frm_left = jnp.concatenate([zc, x[:, :-1]], axis=1)
    frm_right = jnp.concatenate([x[:, 1:], zc], axis=1)
    frm_left = jnp.where(col == 0, 0.0, frm_left)
    frm_right = jnp.where(col == W - 1, 0.0, frm_right)
    rs = x + frm_left + frm_right

    zw = jnp.zeros((C, W), x.dtype)
    frm_up = jnp.concatenate([zw, rs[:, :-W]], axis=1)
    frm_down = jnp.concatenate([rs[:, W:], zw], axis=1)
    s = rs + frm_up + frm_down

    invw = jnp.where((col == 0) | (col == W - 1), 0.5, 1.0 / 3.0)
    invh = jnp.where((row == 0) | (row == H - 1), 0.5, 1.0 / 3.0)
    m = s * (invw * invh)

    y = lax.dot_general(w_ref[...], m, (((1,), (0,)), ((), ())),
                        preferred_element_type=jnp.float32)
    o_ref[0] = x + y + b_ref[...]


def kernel(x, conv_w, conv_b):
    B, C, H, W = x.shape
    x2 = x.reshape(B, C, H * W)
    b2 = conv_b.reshape(C, 1)
    out = pl.pallas_call(
        functools.partial(_dn_kernel, H=H, W=W),
        grid=(B,),
        in_specs=[
            pl.BlockSpec((1, C, H * W), lambda b: (b, 0, 0)),
            pl.BlockSpec((C, C), lambda b: (0, 0)),
            pl.BlockSpec((C, 1), lambda b: (0, 0)),
        ],
        out_specs=pl.BlockSpec((1, C, H * W), lambda b: (b, 0, 0)),
        out_shape=jax.ShapeDtypeStruct((B, C, H * W), x.dtype),
        compiler_params=pltpu.CompilerParams(
            dimension_semantics=("parallel",),
        ),
    )(x2, conv_w, b2)
    return out.reshape(B, C, H, W)


# speedup vs baseline: 9.0802x; 2.0902x over previous
import functools

import jax
import jax.numpy as jnp
from jax import lax
from jax.experimental import pallas as pl
from jax.experimental.pallas import tpu as pltpu


def _dn_kernel(x_ref, w_ref, b_ref, o_ref, *, H, W):
    x = x_ref[0]
    C = x.shape[0]

    zw = jnp.zeros((C, H, 1), x.dtype)
    rs = x + jnp.concatenate([zw, x[:, :, :-1]], axis=2) \
           + jnp.concatenate([x[:, :, 1:], zw], axis=2)

    zh = jnp.zeros((C, 1, W), x.dtype)
    s = rs + jnp.concatenate([zh, rs[:, :-1, :]], axis=1) \
           + jnp.concatenate([rs[:, 1:, :], zh], axis=1)

    hi = lax.broadcasted_iota(jnp.int32, (C, H, W), 1)
    wi = lax.broadcasted_iota(jnp.int32, (C, H, W), 2)
    invh = jnp.where((hi == 0) | (hi == H - 1), 0.5, 1.0 / 3.0)
    invw = jnp.where((wi == 0) | (wi == W - 1), 0.5, 1.0 / 3.0)
    m = s * (invh * invw)

    m2 = m.reshape(C, H * W)
    y2 = lax.dot_general(w_ref[...], m2, (((1,), (0,)), ((), ())),
                         preferred_element_type=jnp.float32)
    y2 = y2 + b_ref[...]
    o_ref[0] = x + y2.reshape(C, H, W)


def kernel(x, conv_w, conv_b):
    B, C, H, W = x.shape
    b2 = conv_b.reshape(C, 1)
    return pl.pallas_call(
        functools.partial(_dn_kernel, H=H, W=W),
        grid=(B,),
        in_specs=[
            pl.BlockSpec((1, C, H, W), lambda b: (b, 0, 0, 0)),
            pl.BlockSpec((C, C), lambda b: (0, 0)),
            pl.BlockSpec((C, 1), lambda b: (0, 0)),
        ],
        out_specs=pl.BlockSpec((1, C, H, W), lambda b: (b, 0, 0, 0)),
        out_shape=jax.ShapeDtypeStruct((B, C, H, W), x.dtype),
        compiler_params=pltpu.CompilerParams(
            dimension_semantics=("parallel",),
        ),
    )(x, conv_w, b2)


# speedup vs baseline: 11.3081x; 1.2454x over previous
import functools

import jax
import jax.numpy as jnp
from jax import lax
from jax.experimental import pallas as pl
from jax.experimental.pallas import tpu as pltpu


def _dn_kernel(x_ref, w_ref, b_ref, tw_ref, inv_ref, o_ref, *, H, W):
    x = x_ref[0]
    C = x.shape[0]

    zh = jnp.zeros((C, 1, W), x.dtype)
    v = x + jnp.concatenate([zh, x[:, :-1, :]], axis=1) \
          + jnp.concatenate([x[:, 1:, :], zh], axis=1)

    v2 = v.reshape(C, H * W)
    t2 = lax.dot_general(w_ref[...], v2, (((1,), (0,)), ((), ())),
                         preferred_element_type=jnp.float32)
    t3 = t2.reshape(C, H, W)

    s = lax.dot_general(t3.reshape(C * H, W), tw_ref[...],
                        (((1,), (0,)), ((), ())),
                        preferred_element_type=jnp.float32).reshape(C, H, W)

    o_ref[0] = x + s * inv_ref[...] + b_ref[...]


def kernel(x, conv_w, conv_b):
    B, C, H, W = x.shape
    f32 = jnp.float32

    i = jnp.arange(W)
    tw = (jnp.abs(i[:, None] - i[None, :]) <= 1).astype(f32)
    ch = jnp.where((jnp.arange(H) == 0) | (jnp.arange(H) == H - 1), 2.0, 3.0)
    cw = jnp.where((i == 0) | (i == W - 1), 2.0, 3.0)
    inv = (1.0 / (ch[:, None] * cw[None, :])).astype(f32)[None]
    b3 = conv_b.reshape(C, 1, 1)

    return pl.pallas_call(
        functools.partial(_dn_kernel, H=H, W=W),
        grid=(B,),
        in_specs=[
            pl.BlockSpec((1, C, H, W), lambda b: (b, 0, 0, 0)),
            pl.BlockSpec((C, C), lambda b: (0, 0)),
            pl.BlockSpec((C, 1, 1), lambda b: (0, 0, 0)),
            pl.BlockSpec((W, W), lambda b: (0, 0)),
            pl.BlockSpec((1, H, W), lambda b: (0, 0, 0)),
        ],
        out_specs=pl.BlockSpec((1, C, H, W), lambda b: (b, 0, 0, 0)),
        out_shape=jax.ShapeDtypeStruct((B, C, H, W), x.dtype),
        compiler_params=pltpu.CompilerParams(
            dimension_semantics=("parallel",),
        ),
    )(x, conv_w, b3, tw, inv)
